# Initial kernel scaffold; baseline (speedup 1.0000x reference)
import jax
import jax.numpy as jnp
from jax.experimental import pallas as pl


def kernel(token_ids, weight):
    raise NotImplementedError("write your pallas kernel here")


# speedup vs baseline: 4.8075x; 4.8075x over previous
import functools

import jax
import jax.numpy as jnp
from jax import lax
from jax.experimental import pallas as pl
from jax.experimental.pallas import tpu as pltpu
from jax.experimental.pallas import tpu_sc as plsc

NUM_EMB = 1000000
DIM = 32
BATCH = 16384
SEQ = 200
B = BATCH * SEQ

NC = 2
NS = 16
NW = NC * NS
B_PER_W = B // NW
CHUNK = 1024
N_CHUNKS = B_PER_W // CHUNK

assert B % NW == 0 and B_PER_W % CHUNK == 0


def _emb_body(idx_hbm, table_hbm, out_hbm, idx_v, rows_v, gsem):
    wid = lax.axis_index("s") * NC + lax.axis_index("c")
    base = wid * B_PER_W

    @pl.loop(0, N_CHUNKS)
    def _(g):
        off = base + g * CHUNK
        pltpu.sync_copy(idx_hbm.at[pl.ds(off, CHUNK)], idx_v)
        pltpu.async_copy(table_hbm.at[idx_v], rows_v, gsem).wait()
        pltpu.sync_copy(rows_v, out_hbm.at[pl.ds(off, CHUNK)])


@jax.jit
def _emb_lookup(idx_flat, weight):
    mesh = plsc.VectorSubcoreMesh(core_axis_name="c", subcore_axis_name="s")
    return pl.kernel(
        _emb_body,
        out_type=jax.ShapeDtypeStruct((B, DIM), jnp.float32),
        mesh=mesh,
        scratch_types=[
            pltpu.VMEM((CHUNK,), jnp.int32),
            pltpu.VMEM((CHUNK, DIM), jnp.float32),
            pltpu.SemaphoreType.DMA,
        ],
        compiler_params=pltpu.CompilerParams(use_tc_tiling_on_sc=False),
    )(idx_flat, weight)


def kernel(token_ids, weight):
    idx_flat = token_ids.reshape(-1).astype(jnp.int32)
    out = _emb_lookup(idx_flat, weight)
    return out.reshape(BATCH, SEQ, DIM)


# speedup vs baseline: 5.0356x; 1.0474x over previous
import functools

import jax
import jax.numpy as jnp
from jax import lax
from jax.experimental import pallas as pl
from jax.experimental.pallas import tpu as pltpu
from jax.experimental.pallas import tpu_sc as plsc

NUM_EMB = 1000000
DIM = 32
BATCH = 16384
SEQ = 200
B = BATCH * SEQ

NC = 2
NS = 16
NW = NC * NS
B_PER_W = B // NW
CHUNK = 1024
N_CHUNKS = B_PER_W // CHUNK

assert B % NW == 0 and B_PER_W % CHUNK == 0


def _emb_body(idx_hbm, table_hbm, out_hbm,
              idx_v0, idx_v1, rows_v0, rows_v1,
              isem0, isem1, gsem0, gsem1, osem0, osem1):
    wid = lax.axis_index("s") * NC + lax.axis_index("c")
    base = wid * B_PER_W

    idx_v = (idx_v0, idx_v1)
    rows_v = (rows_v0, rows_v1)
    isem = (isem0, isem1)
    gsem = (gsem0, gsem1)
    osem = (osem0, osem1)

    def ichunk(g):
        return idx_hbm.at[pl.ds(base + g * CHUNK, CHUNK)]

    def ochunk(g):
        return out_hbm.at[pl.ds(base + g * CHUNK, CHUNK)]

    pltpu.async_copy(ichunk(0), idx_v[0], isem[0])
    pltpu.async_copy(ichunk(1), idx_v[1], isem[1])
    pltpu.make_async_copy(ichunk(0), idx_v[0], isem[0]).wait()
    pltpu.async_copy(table_hbm.at[idx_v[0]], rows_v[0], gsem[0])

    @pl.loop(0, N_CHUNKS, step=2)
    def _(g0):
        for b in range(2):
            g = g0 + b
            nb = 1 - b
            pltpu.make_async_copy(
                table_hbm.at[idx_v[b]], rows_v[b], gsem[b]).wait()

            @pl.when(g + 2 < N_CHUNKS)
            def _():
                pltpu.async_copy(ichunk(g + 2), idx_v[b], isem[b])

            @pl.when(g + 1 < N_CHUNKS)
            def _():
                pltpu.make_async_copy(
                    ichunk(g + 1), idx_v[nb], isem[nb]).wait()

                @pl.when(g >= 1)
                def _():
                    pltpu.make_async_copy(
                        rows_v[nb], ochunk(g - 1), osem[nb]).wait()

                pltpu.async_copy(
                    table_hbm.at[idx_v[nb]], rows_v[nb], gsem[nb])

            pltpu.async_copy(rows_v[b], ochunk(g), osem[b])

    pltpu.make_async_copy(
        rows_v[0], ochunk(N_CHUNKS - 2), osem[0]).wait()
    pltpu.make_async_copy(
        rows_v[1], ochunk(N_CHUNKS - 1), osem[1]).wait()


@jax.jit
def _emb_lookup(idx_flat, weight):
    mesh = plsc.VectorSubcoreMesh(core_axis_name="c", subcore_axis_name="s")
    return pl.kernel(
        _emb_body,
        out_type=jax.ShapeDtypeStruct((B, DIM), jnp.float32),
        mesh=mesh,
        scratch_types=[
            pltpu.VMEM((CHUNK,), jnp.int32),
            pltpu.VMEM((CHUNK,), jnp.int32),
            pltpu.VMEM((CHUNK, DIM), jnp.float32),
            pltpu.VMEM((CHUNK, DIM), jnp.float32),
            pltpu.SemaphoreType.DMA,
            pltpu.SemaphoreType.DMA,
            pltpu.SemaphoreType.DMA,
            pltpu.SemaphoreType.DMA,
            pltpu.SemaphoreType.DMA,
            pltpu.SemaphoreType.DMA,
        ],
        compiler_params=pltpu.CompilerParams(use_tc_tiling_on_sc=False),
    )(idx_flat, weight)


def kernel(token_ids, weight):
    idx_flat = token_ids.reshape(-1).astype(jnp.int32)
    out = _emb_lookup(idx_flat, weight)
    return out.reshape(BATCH, SEQ, DIM)


# speedup vs baseline: 5.0535x; 1.0036x over previous
import functools

import jax
import jax.numpy as jnp
from jax import lax
from jax.experimental import pallas as pl
from jax.experimental.pallas import tpu as pltpu
from jax.experimental.pallas import tpu_sc as plsc

NUM_EMB = 1000000
DIM = 32
BATCH = 16384
SEQ = 200
B = BATCH * SEQ

NC = 2
NS = 16
NW = NC * NS
B_PER_W = B // NW
CHUNK = 512
NBUF = 4
N_CHUNKS = B_PER_W // CHUNK

assert B % NW == 0 and B_PER_W % CHUNK == 0 and N_CHUNKS % NBUF == 0


def _emb_body(idx_hbm, table_hbm, out_hbm, *scratch):
    idx_v = scratch[:NBUF]
    rows_v = scratch[NBUF:2 * NBUF]
    isem = scratch[2 * NBUF:3 * NBUF]
    gsem = scratch[3 * NBUF:4 * NBUF]
    osem = scratch[4 * NBUF:5 * NBUF]

    wid = lax.axis_index("s") * NC + lax.axis_index("c")
    base = wid * B_PER_W

    def ichunk(g):
        return idx_hbm.at[pl.ds(base + g * CHUNK, CHUNK)]

    def ochunk(g):
        return out_hbm.at[pl.ds(base + g * CHUNK, CHUNK)]

    for h in range(NBUF):
        pltpu.async_copy(ichunk(h), idx_v[h], isem[h])
    for h in range(NBUF - 1):
        pltpu.make_async_copy(ichunk(h), idx_v[h], isem[h]).wait()
        pltpu.async_copy(table_hbm.at[idx_v[h]], rows_v[h], gsem[h])

    @pl.loop(0, N_CHUNKS, step=NBUF)
    def _(g0):
        for b in range(NBUF):
            g = g0 + b
            pb = (b + NBUF - 1) % NBUF
            pltpu.make_async_copy(
                table_hbm.at[idx_v[b]], rows_v[b], gsem[b]).wait()

            @pl.when(g + NBUF < N_CHUNKS)
            def _():
                pltpu.async_copy(ichunk(g + NBUF), idx_v[b], isem[b])

            @pl.when(g + NBUF - 1 < N_CHUNKS)
            def _():
                pltpu.make_async_copy(
                    ichunk(g + NBUF - 1), idx_v[pb], isem[pb]).wait()

                @pl.when(g >= 1)
                def _():
                    pltpu.make_async_copy(
                        rows_v[pb], ochunk(g - 1), osem[pb]).wait()

                pltpu.async_copy(
                    table_hbm.at[idx_v[pb]], rows_v[pb], gsem[pb])

            pltpu.async_copy(rows_v[b], ochunk(g), osem[b])

    for h in range(NBUF):
        g = N_CHUNKS - NBUF + h
        pltpu.make_async_copy(
            rows_v[g % NBUF], ochunk(g), osem[g % NBUF]).wait()


@jax.jit
def _emb_lookup(idx_flat, weight):
    mesh = plsc.VectorSubcoreMesh(core_axis_name="c", subcore_axis_name="s")
    return pl.kernel(
        _emb_body,
        out_type=jax.ShapeDtypeStruct((B, DIM), jnp.float32),
        mesh=mesh,
        scratch_types=(
            [pltpu.VMEM((CHUNK,), jnp.int32) for _ in range(NBUF)]
            + [pltpu.VMEM((CHUNK, DIM), jnp.float32) for _ in range(NBUF)]
            + [pltpu.SemaphoreType.DMA for _ in range(3 * NBUF)]
        ),
        compiler_params=pltpu.CompilerParams(use_tc_tiling_on_sc=False),
    )(idx_flat, weight)


def kernel(token_ids, weight):
    idx_flat = token_ids.reshape(-1).astype(jnp.int32)
    out = _emb_lookup(idx_flat, weight)
    return out.reshape(BATCH, SEQ, DIM)
